# Initial kernel scaffold; baseline (speedup 1.0000x reference)
import jax
import jax.numpy as jnp
from jax.experimental import pallas as pl


def kernel(x, edge_index_0, edge_index_1, rel_logits, W_l0, b_l0, W_r0, W_l1, b_l1, W_r1):
    raise NotImplementedError("write your pallas kernel here")


# speedup vs baseline: 3.2383x; 3.2383x over previous
import functools

import jax
import jax.numpy as jnp
from jax import lax
from jax.experimental import pallas as pl
from jax.experimental.pallas import tpu as pltpu
from jax.experimental.pallas import tpu_sc as plsc

N = 10000
D = 128
E = 160000
HD = D // 2
WP = HD + 8
NC = 2
NS = 16
CH = 128
EC = E // NS
NCH = -(-EC // CH)
EPAD = NCH * CH
RPAD = 10240
RT = RPAD // NS
RC = RT // CH
BR = 1000


def _sc_agg_body(xlo_hbm, xhi_hbm, src_hbm, dst_hbm, zeros_hbm, out_hbm,
                 src_v, dst_v, rows_v, accum, gsem):
    c = lax.axis_index("c")
    s = lax.axis_index("s")
    pltpu.sync_copy(src_hbm.at[c, s], src_v)
    pltpu.sync_copy(dst_hbm.at[c, s], dst_v)
    for p, x_hbm in ((0, xlo_hbm), (1, xhi_hbm)):
        for k in range(RC):
            pltpu.sync_copy(zeros_hbm, accum.at[pl.ds(s * RT + k * CH, CH)])
        plsc.subcore_barrier()

        def chunk(j, carry):
            buf = rows_v.at[j % 2]
            pltpu.async_copy(x_hbm.at[src_v.at[j]], buf, gsem).wait()
            pltpu.sync_copy(buf, accum.at[dst_v.at[j]], add=True)
            return carry

        lax.fori_loop(0, NCH, chunk, 0)
        plsc.subcore_barrier()
        for k in range(RC):
            r0 = s * RT + k * CH
            pltpu.sync_copy(accum.at[pl.ds(r0, CH)], out_hbm.at[c, p, pl.ds(r0, CH)])


_sc_agg = functools.partial(
    pl.kernel,
    out_type=jax.ShapeDtypeStruct((NC, 2, RPAD, WP), jnp.float32),
    mesh=plsc.VectorSubcoreMesh(core_axis_name="c", subcore_axis_name="s"),
    scratch_types=[
        pltpu.VMEM((NCH, CH), jnp.int32),
        pltpu.VMEM((NCH, CH), jnp.int32),
        pltpu.VMEM((2, CH, WP), jnp.float32),
        pltpu.VMEM_SHARED((RPAD, WP), jnp.float32),
        pltpu.SemaphoreType.DMA,
    ],
    compiler_params=pltpu.CompilerParams(use_tc_tiling_on_sc=False),
)(_sc_agg_body)


def _tc_body(rl_ref, a0l_ref, a0h_ref, a1l_ref, a1h_ref, x_ref,
             wl0_ref, wl1_ref, wr0_ref, wr1_ref, b_ref, out_ref, rw_ref):
    rl = rl_ref[0, :]
    e = jnp.exp(rl - jnp.max(rl))
    w = e / jnp.sum(e)
    rw_ref[0, :] = w
    a0l = a0l_ref[...]
    a0h = a0h_ref[...]
    a1l = a1l_ref[...]
    a1h = a1h_ref[...]
    c0 = jnp.maximum(a0l[:, HD:HD + 1], 1.0)
    c1 = jnp.maximum(a1l[:, HD:HD + 1], 1.0)
    wl0 = wl0_ref[...]
    wl1 = wl1_ref[...]
    h0 = (jnp.dot(a0l[:, :HD] / c0, wl0[:HD], preferred_element_type=jnp.float32)
          + jnp.dot(a0h[:, :HD] / c0, wl0[HD:], preferred_element_type=jnp.float32))
    h1 = (jnp.dot(a1l[:, :HD] / c1, wl1[:HD], preferred_element_type=jnp.float32)
          + jnp.dot(a1h[:, :HD] / c1, wl1[HD:], preferred_element_type=jnp.float32))
    wr = wr0_ref[...] * w[0:1] + wr1_ref[...] * w[1:2]
    hr = jnp.dot(x_ref[...], wr, preferred_element_type=jnp.float32)
    bias = b_ref[0:1, :] * w[0:1] + b_ref[1:2, :] * w[1:2]
    out_ref[...] = h0 * w[0:1] + h1 * w[1:2] + hr + bias


_tc_combine = pl.pallas_call(
    _tc_body,
    grid=(N // BR,),
    in_specs=[
        pl.BlockSpec((1, 2), lambda i: (0, 0)),
        pl.BlockSpec((BR, WP), lambda i: (i, 0)),
        pl.BlockSpec((BR, WP), lambda i: (i, 0)),
        pl.BlockSpec((BR, WP), lambda i: (i, 0)),
        pl.BlockSpec((BR, WP), lambda i: (i, 0)),
        pl.BlockSpec((BR, D), lambda i: (i, 0)),
        pl.BlockSpec((D, D), lambda i: (0, 0)),
        pl.BlockSpec((D, D), lambda i: (0, 0)),
        pl.BlockSpec((D, D), lambda i: (0, 0)),
        pl.BlockSpec((D, D), lambda i: (0, 0)),
        pl.BlockSpec((2, D), lambda i: (0, 0)),
    ],
    out_specs=[
        pl.BlockSpec((BR, D), lambda i: (i, 0)),
        pl.BlockSpec((1, 2), lambda i: (0, 0)),
    ],
    out_shape=[
        jax.ShapeDtypeStruct((N, D), jnp.float32),
        jax.ShapeDtypeStruct((1, 2), jnp.float32),
    ],
)


def _prep_idx(edge_index):
    src = edge_index[0].reshape(NS, EC)
    dst = edge_index[1].reshape(NS, EC)
    pad = EPAD - EC
    src = jnp.pad(src, ((0, 0), (0, pad))).reshape(NS, NCH, CH)
    dst = jnp.pad(dst, ((0, 0), (0, pad)), constant_values=N).reshape(NS, NCH, CH)
    return src, dst


def kernel(x, edge_index_0, edge_index_1, rel_logits,
           W_l0, b_l0, W_r0, W_l1, b_l1, W_r1):
    ones = jnp.ones((N, WP - HD), jnp.float32)
    xlo = jnp.concatenate([x[:, :HD], ones], axis=1)
    xhi = jnp.concatenate([x[:, HD:], ones], axis=1)
    s0, d0 = _prep_idx(edge_index_0)
    s1, d1 = _prep_idx(edge_index_1)
    src_all = jnp.stack([s0, s1])
    dst_all = jnp.stack([d0, d1])
    zeros = jnp.zeros((CH, WP), jnp.float32)
    agg = _sc_agg(xlo, xhi, src_all, dst_all, zeros)
    new_x, rw = _tc_combine(rel_logits.reshape(1, 2),
                            agg[0, 0, :N], agg[0, 1, :N],
                            agg[1, 0, :N], agg[1, 1, :N],
                            x, W_l0, W_l1, W_r0, W_r1,
                            jnp.stack([b_l0, b_l1]))
    return new_x, rw.reshape(2)


# speedup vs baseline: 3.7074x; 1.1449x over previous
import functools

import jax
import jax.numpy as jnp
from jax import lax
from jax.experimental import pallas as pl
from jax.experimental.pallas import tpu as pltpu
from jax.experimental.pallas import tpu_sc as plsc

N = 10000
D = 128
E = 160000
HD = D // 2
WP = HD + 8
NC = 2
NS = 16
CH = 128
EC = E // NS
NCH = -(-EC // CH)
EPAD = NCH * CH
RPAD = 10240
RT = RPAD // NS
RC = RT // CH
BR = 1000


def _sc_agg_body(xlo_hbm, xhi_hbm, src_hbm, dst_hbm, zeros_hbm, out_hbm,
                 src_v, dst_v, rows_v, accum, gsem, ssem):
    c = lax.axis_index("c")
    s = lax.axis_index("s")
    pltpu.sync_copy(src_hbm.at[c, s], src_v)
    pltpu.sync_copy(dst_hbm.at[c, s], dst_v)
    for p, x_hbm in ((0, xlo_hbm), (1, xhi_hbm)):
        for k in range(RC):
            pltpu.sync_copy(zeros_hbm, accum.at[pl.ds(s * RT + k * CH, CH)])
        plsc.subcore_barrier()
        pltpu.async_copy(x_hbm.at[src_v.at[0]], rows_v.at[0], gsem)

        def chunk(j, carry):
            buf = rows_v.at[j % 2]
            pltpu.make_async_copy(x_hbm.at[src_v.at[j]], buf, gsem).wait()
            pltpu.async_copy(buf, accum.at[dst_v.at[j]], ssem, add=True)

            @pl.when(j >= 1)
            def _():
                pltpu.make_async_copy(rows_v.at[(j + 1) % 2],
                                      accum.at[dst_v.at[j - 1]], ssem).wait()

            @pl.when(j + 1 < NCH)
            def _():
                pltpu.async_copy(x_hbm.at[src_v.at[j + 1]],
                                 rows_v.at[(j + 1) % 2], gsem)
            return carry

        lax.fori_loop(0, NCH, chunk, 0)
        pltpu.make_async_copy(rows_v.at[(NCH - 1) % 2],
                              accum.at[dst_v.at[NCH - 1]], ssem).wait()
        plsc.subcore_barrier()
        for k in range(RC):
            r0 = s * RT + k * CH
            pltpu.sync_copy(accum.at[pl.ds(r0, CH)], out_hbm.at[c, p, pl.ds(r0, CH)])


_sc_agg = functools.partial(
    pl.kernel,
    out_type=jax.ShapeDtypeStruct((NC, 2, RPAD, WP), jnp.float32),
    mesh=plsc.VectorSubcoreMesh(core_axis_name="c", subcore_axis_name="s"),
    scratch_types=[
        pltpu.VMEM((NCH, CH), jnp.int32),
        pltpu.VMEM((NCH, CH), jnp.int32),
        pltpu.VMEM((2, CH, WP), jnp.float32),
        pltpu.VMEM_SHARED((RPAD, WP), jnp.float32),
        pltpu.SemaphoreType.DMA,
        pltpu.SemaphoreType.DMA,
    ],
    compiler_params=pltpu.CompilerParams(use_tc_tiling_on_sc=False),
)(_sc_agg_body)


def _tc_body(rl_ref, a0l_ref, a0h_ref, a1l_ref, a1h_ref, x_ref,
             wl0_ref, wl1_ref, wr0_ref, wr1_ref, b_ref, out_ref, rw_ref):
    rl = rl_ref[0, :]
    e = jnp.exp(rl - jnp.max(rl))
    w = e / jnp.sum(e)
    rw_ref[0, :] = w
    a0l = a0l_ref[0, 0]
    a0h = a0h_ref[0, 0]
    a1l = a1l_ref[0, 0]
    a1h = a1h_ref[0, 0]
    c0 = jnp.maximum(a0l[:, HD:HD + 1], 1.0)
    c1 = jnp.maximum(a1l[:, HD:HD + 1], 1.0)
    wl0 = wl0_ref[...]
    wl1 = wl1_ref[...]
    h0 = (jnp.dot(a0l[:, :HD] / c0, wl0[:HD], preferred_element_type=jnp.float32)
          + jnp.dot(a0h[:, :HD] / c0, wl0[HD:], preferred_element_type=jnp.float32))
    h1 = (jnp.dot(a1l[:, :HD] / c1, wl1[:HD], preferred_element_type=jnp.float32)
          + jnp.dot(a1h[:, :HD] / c1, wl1[HD:], preferred_element_type=jnp.float32))
    wr = wr0_ref[...] * w[0:1] + wr1_ref[...] * w[1:2]
    hr = jnp.dot(x_ref[...], wr, preferred_element_type=jnp.float32)
    bias = b_ref[0:1, :] * w[0:1] + b_ref[1:2, :] * w[1:2]
    out_ref[...] = h0 * w[0:1] + h1 * w[1:2] + hr + bias


_tc_combine = pl.pallas_call(
    _tc_body,
    grid=(N // BR,),
    in_specs=[
        pl.BlockSpec((1, 2), lambda i: (0, 0)),
        pl.BlockSpec((1, 1, BR, WP), lambda i: (0, 0, i, 0)),
        pl.BlockSpec((1, 1, BR, WP), lambda i: (0, 1, i, 0)),
        pl.BlockSpec((1, 1, BR, WP), lambda i: (1, 0, i, 0)),
        pl.BlockSpec((1, 1, BR, WP), lambda i: (1, 1, i, 0)),
        pl.BlockSpec((BR, D), lambda i: (i, 0)),
        pl.BlockSpec((D, D), lambda i: (0, 0)),
        pl.BlockSpec((D, D), lambda i: (0, 0)),
        pl.BlockSpec((D, D), lambda i: (0, 0)),
        pl.BlockSpec((D, D), lambda i: (0, 0)),
        pl.BlockSpec((2, D), lambda i: (0, 0)),
    ],
    out_specs=[
        pl.BlockSpec((BR, D), lambda i: (i, 0)),
        pl.BlockSpec((1, 2), lambda i: (0, 0)),
    ],
    out_shape=[
        jax.ShapeDtypeStruct((N, D), jnp.float32),
        jax.ShapeDtypeStruct((1, 2), jnp.float32),
    ],
)


def _prep_idx(edge_index):
    src = edge_index[0].reshape(NS, EC)
    dst = edge_index[1].reshape(NS, EC)
    pad = EPAD - EC
    src = jnp.pad(src, ((0, 0), (0, pad))).reshape(NS, NCH, CH)
    dst = jnp.pad(dst, ((0, 0), (0, pad)), constant_values=N).reshape(NS, NCH, CH)
    return src, dst


def kernel(x, edge_index_0, edge_index_1, rel_logits,
           W_l0, b_l0, W_r0, W_l1, b_l1, W_r1):
    ones = jnp.ones((N, WP - HD), jnp.float32)
    xlo = jnp.concatenate([x[:, :HD], ones], axis=1)
    xhi = jnp.concatenate([x[:, HD:], ones], axis=1)
    s0, d0 = _prep_idx(edge_index_0)
    s1, d1 = _prep_idx(edge_index_1)
    src_all = jnp.stack([s0, s1])
    dst_all = jnp.stack([d0, d1])
    zeros = jnp.zeros((CH, WP), jnp.float32)
    agg = _sc_agg(xlo, xhi, src_all, dst_all, zeros)
    new_x, rw = _tc_combine(rel_logits.reshape(1, 2),
                            agg, agg, agg, agg,
                            x, W_l0, W_l1, W_r0, W_r1,
                            jnp.stack([b_l0, b_l1]))
    return new_x, rw.reshape(2)
